# baseline (speedup 1.0000x reference)
import functools

import jax
import jax.numpy as jnp
from jax import lax
from jax.experimental import pallas as pl
from jax.experimental.pallas import tpu as pltpu
from jax.experimental.pallas import tpu_sc as plsc

NC = 2
NS = 16
NW = NC * NS

C = 128
K = 8


@functools.partial(jax.jit, static_argnames=("n_groups",))
def _sc_gather(W, idx2d, n_groups):
    btot = idx2d.shape[0]
    n_rows = btot // C
    d = W.shape[1]
    rows_per_w = n_rows // NW

    mesh = plsc.VectorSubcoreMesh(core_axis_name="c", subcore_axis_name="s")

    @functools.partial(
        pl.kernel,
        out_type=jax.ShapeDtypeStruct((btot, d), jnp.float32),
        mesh=mesh,
        scratch_types=[
            pltpu.VMEM((K * C,), jnp.int32),
            pltpu.VMEM((K * C, d), jnp.float32),
            pltpu.SemaphoreType.DMA((K,)),
            pltpu.SemaphoreType.DMA((K,)),
        ],
        compiler_params=pltpu.CompilerParams(use_tc_tiling_on_sc=False),
    )
    def body(table_hbm, idx_hbm, out_hbm, idx_v, rows_v, gsem, osem):
        wid = lax.axis_index("s") * NC + lax.axis_index("c")
        row_base = wid * rows_per_w

        def group(g, carry):
            roff = row_base + g * K
            pltpu.sync_copy(idx_hbm.at[pl.ds(roff * C, K * C)], idx_v)
            pltpu.make_async_copy(
                table_hbm.at[idx_v], rows_v, gsem.at[0]
            ).start()
            pltpu.make_async_copy(
                table_hbm.at[idx_v], rows_v, gsem.at[0]
            ).wait()
            pltpu.make_async_copy(
                rows_v, out_hbm.at[pl.ds(roff * C, K * C)], osem.at[0]
            ).start()
            pltpu.make_async_copy(
                rows_v, out_hbm.at[pl.ds(roff * C, K * C)], osem.at[0]
            ).wait()
            return carry

        lax.fori_loop(0, n_groups, group, 0)

    return body(W, idx2d)


def kernel(token_ids, W):
    b, h = token_ids.shape
    d = W.shape[1]
    idx = token_ids.reshape(-1).astype(jnp.int32)
    btot = idx.shape[0]
    idx2d = idx
    n_groups = btot // (NW * C * K)
    out = _sc_gather(W, idx2d, n_groups)
    return out.reshape(b, h, d)


# speedup vs baseline: 1.0440x; 1.0440x over previous
import functools

import jax
import jax.numpy as jnp
from jax import lax
from jax.experimental import pallas as pl
from jax.experimental.pallas import tpu as pltpu
from jax.experimental.pallas import tpu_sc as plsc

NC = 2
NS = 16
NW = NC * NS

C = 128
K = 8


@functools.partial(jax.jit, static_argnames=("n_groups",))
def _sc_gather(W, idx2d, n_groups):
    btot = idx2d.shape[0]
    n_rows = btot // C
    d = W.shape[1]
    rows_per_w = n_rows // NW

    mesh = plsc.VectorSubcoreMesh(core_axis_name="c", subcore_axis_name="s")

    @functools.partial(
        pl.kernel,
        out_type=jax.ShapeDtypeStruct((btot, d), jnp.float32),
        mesh=mesh,
        scratch_types=[
            pltpu.VMEM((K * C,), jnp.int32),
            pltpu.VMEM((K * C, d), jnp.float32),
            pltpu.SemaphoreType.DMA((K,)),
            pltpu.SemaphoreType.DMA((K,)),
        ],
        compiler_params=pltpu.CompilerParams(use_tc_tiling_on_sc=False),
    )
    def body(table_hbm, idx_hbm, out_hbm, idx_v, rows_v, gsem, osem):
        wid = lax.axis_index("s") * NC + lax.axis_index("c")
        row_base = wid * rows_per_w

        def group(g, carry):
            roff = row_base + g * K
            pltpu.sync_copy(idx_hbm.at[pl.ds(roff * C, K * C)], idx_v)
            pltpu.make_async_copy(
                table_hbm.at[idx_v], rows_v, gsem.at[0]
            ).start()
            pltpu.make_async_copy(
                table_hbm.at[idx_v], rows_v, gsem.at[0]
            ).wait()
            pltpu.make_async_copy(
                rows_v, out_hbm.at[pl.ds(roff * C, K * C)], osem.at[0]
            ).start()
            pltpu.make_async_copy(
                rows_v, out_hbm.at[pl.ds(roff * C, K * C)], osem.at[0]
            ).wait()
            return carry

        lax.fori_loop(0, n_groups, group, 0)

    return body(W, idx2d)


def kernel(token_ids, W):
    b, h = token_ids.shape
    d = W.shape[1]
    idx = token_ids.T.reshape(-1).astype(jnp.int32)
    btot = idx.shape[0]
    n_groups = btot // (NW * C * K)
    out = _sc_gather(W, idx, n_groups)
    return out.reshape(h, b, d).transpose(1, 0, 2)


# speedup vs baseline: 1.0528x; 1.0085x over previous
import functools

import jax
import jax.numpy as jnp
from jax import lax
from jax.experimental import pallas as pl
from jax.experimental.pallas import tpu as pltpu
from jax.experimental.pallas import tpu_sc as plsc

NC = 2
NS = 16
NW = NC * NS

C = 512
NB = 2


@functools.partial(jax.jit, static_argnames=("n_pairs",))
def _sc_gather(W, idx, n_pairs):
    btot = idx.shape[0]
    d = W.shape[1]

    mesh = plsc.VectorSubcoreMesh(core_axis_name="c", subcore_axis_name="s")

    @functools.partial(
        pl.kernel,
        out_type=jax.ShapeDtypeStruct((btot, d), jnp.float32),
        mesh=mesh,
        scratch_types=[
            pltpu.VMEM((NB, C), jnp.int32),
            pltpu.VMEM((NB, C, d), jnp.float32),
            pltpu.SemaphoreType.DMA((NB,)),
            pltpu.SemaphoreType.DMA((NB,)),
        ],
        compiler_params=pltpu.CompilerParams(use_tc_tiling_on_sc=False),
    )
    def body(table_hbm, idx_hbm, out_hbm, idx_v, rows_v, gsem, osem):
        wid = lax.axis_index("s") * NC + lax.axis_index("c")
        base = wid * (n_pairs * NB * C)

        def pair(p, carry):
            off = base + p * (NB * C)
            for b in range(NB):
                pltpu.sync_copy(idx_hbm.at[pl.ds(off + b * C, C)], idx_v.at[b])
                pltpu.make_async_copy(
                    table_hbm.at[idx_v.at[b]], rows_v.at[b], gsem.at[b]
                ).start()
            for b in range(NB):
                pltpu.make_async_copy(
                    table_hbm.at[idx_v.at[b]], rows_v.at[b], gsem.at[b]
                ).wait()
                pltpu.make_async_copy(
                    rows_v.at[b], out_hbm.at[pl.ds(off + b * C, C)], osem.at[b]
                ).start()
            for b in range(NB):
                pltpu.make_async_copy(
                    rows_v.at[b], out_hbm.at[pl.ds(off + b * C, C)], osem.at[b]
                ).wait()
            return carry

        lax.fori_loop(0, n_pairs, pair, 0)

    return body(W, idx)


def kernel(token_ids, W):
    b, h = token_ids.shape
    d = W.shape[1]
    idx = token_ids.T.reshape(-1).astype(jnp.int32)
    btot = idx.shape[0]
    n_pairs = btot // (NW * NB * C)
    out = _sc_gather(W, idx, n_pairs)
    return out.reshape(h, b, d).transpose(1, 0, 2)
